# Initial kernel scaffold; baseline (speedup 1.0000x reference)
import jax
import jax.numpy as jnp
from jax.experimental import pallas as pl


def kernel(cat_ids, position_ids, shape_ids, word_table, pos_table, shape_table):
    raise NotImplementedError("write your pallas kernel here")


# speedup vs baseline: 3.0414x; 3.0414x over previous
import functools

import jax
import jax.numpy as jnp
from jax import lax
from jax.experimental import pallas as pl
from jax.experimental.pallas import tpu as pltpu
from jax.experimental.pallas import tpu_sc as plsc

_L = 16


def _make_sc_embed(N, D, V, P, S):
    info = plsc.get_sparse_core_info()
    NC, NS = info.num_cores, info.num_subcores
    NW = NC * NS
    assert N % NW == 0
    tok_w = N // NW
    CH = 128
    assert tok_w % CH == 0
    n_chunks = tok_w // CH

    mesh = plsc.VectorSubcoreMesh(core_axis_name="c", subcore_axis_name="s")

    @functools.partial(
        pl.kernel,
        mesh=mesh,
        compiler_params=pltpu.CompilerParams(
            use_tc_tiling_on_sc=False, needs_layout_passes=False),
        out_type=jax.ShapeDtypeStruct((N, D), jnp.float32),
        scratch_types=[
            pltpu.VMEM((CH,), jnp.int32),
            pltpu.VMEM((CH,), jnp.int32),
            pltpu.VMEM((CH,), jnp.int32),
            pltpu.VMEM((CH, D), jnp.float32),
            pltpu.VMEM((CH, D), jnp.float32),
            pltpu.VMEM((CH, D), jnp.float32),
            pltpu.SemaphoreType.DMA,
        ],
    )
    def sc_embed(cat_h, pos_h, shp_h, wtab_h, ptab_h, stab_h, out_h,
                 cat_v, pos_v, shp_v, w_v, p_v, s_v, sem):
        wid = lax.axis_index("s") * NC + lax.axis_index("c")
        base = wid * tok_w

        def chunk_body(g, carry):
            tok0 = base + g * CH
            pltpu.sync_copy(cat_h.at[pl.ds(tok0, CH)], cat_v)
            pltpu.sync_copy(pos_h.at[pl.ds(tok0, CH)], pos_v)
            pltpu.sync_copy(shp_h.at[pl.ds(tok0, CH)], shp_v)
            cw = pltpu.async_copy(wtab_h.at[cat_v], w_v, sem)
            cp = pltpu.async_copy(ptab_h.at[pos_v], p_v, sem)
            cs = pltpu.async_copy(stab_h.at[shp_v], s_v, sem)
            cw.wait()
            cp.wait()
            cs.wait()

            def fix_grp(i, c):
                ids = cat_v[pl.ds(i * _L, _L)]
                msk = ids == 0
                nbad = jnp.max(msk.astype(jnp.int32))

                @pl.when(nbad > 0)
                def _():
                    rows = lax.iota(jnp.int32, _L) + i * _L
                    zeros = jnp.zeros((_L,), jnp.float32)
                    for col in range(D):
                        plsc.store_scatter(
                            w_v, [rows, jnp.full((_L,), col, jnp.int32)],
                            zeros, mask=msk)
                return c

            lax.fori_loop(0, CH // _L, fix_grp, 0)

            def tok_body(t, c):
                for j in range(D // _L):
                    sl = (t, pl.ds(j * _L, _L))
                    p_v[sl] = w_v[sl] + p_v[sl] + s_v[sl]
                return c

            lax.fori_loop(0, CH, tok_body, 0)
            pltpu.sync_copy(p_v, out_h.at[pl.ds(tok0, CH)])
            return carry

        lax.fori_loop(0, n_chunks, chunk_body, 0)

    return sc_embed


def kernel(cat_ids, position_ids, shape_ids, word_table, pos_table, shape_table):
    B, L = cat_ids.shape
    V, D = word_table.shape
    N = B * L
    sc_embed = _make_sc_embed(N, D, V, pos_table.shape[0], shape_table.shape[0])
    out = sc_embed(
        cat_ids.reshape(N),
        position_ids.reshape(N),
        shape_ids.reshape(N),
        word_table,
        pos_table,
        shape_table,
    )
    return out.reshape(B, L, D)


# speedup vs baseline: 3.3362x; 1.0969x over previous
import functools

import jax
import jax.numpy as jnp
from jax import lax
from jax.experimental import pallas as pl
from jax.experimental.pallas import tpu as pltpu
from jax.experimental.pallas import tpu_sc as plsc

_L = 16


def _make_sc_embed(N, D):
    info = plsc.get_sparse_core_info()
    NC, NS = info.num_cores, info.num_subcores
    NW = NC * NS
    assert N % NW == 0
    tok_w = N // NW
    CH = 128
    assert tok_w % (2 * CH) == 0
    n_chunks = tok_w // CH

    mesh = plsc.VectorSubcoreMesh(core_axis_name="c", subcore_axis_name="s")

    @functools.partial(
        pl.kernel,
        mesh=mesh,
        compiler_params=pltpu.CompilerParams(
            use_tc_tiling_on_sc=False, needs_layout_passes=False),
        out_type=jax.ShapeDtypeStruct((N, D), jnp.float32),
        scratch_types=[
            pltpu.VMEM((CH,), jnp.int32),
            pltpu.VMEM((CH,), jnp.int32),
            pltpu.VMEM((CH,), jnp.int32),
            pltpu.VMEM((CH,), jnp.int32),
            pltpu.VMEM((CH,), jnp.int32),
            pltpu.VMEM((CH,), jnp.int32),
            pltpu.VMEM((CH, D), jnp.float32),
            pltpu.VMEM((CH, D), jnp.float32),
            pltpu.VMEM((CH, D), jnp.float32),
            pltpu.VMEM((CH, D), jnp.float32),
            pltpu.VMEM((CH, D), jnp.float32),
            pltpu.VMEM((CH, D), jnp.float32),
            pltpu.SemaphoreType.DMA,
            pltpu.SemaphoreType.DMA,
            pltpu.SemaphoreType.DMA,
            pltpu.SemaphoreType.DMA,
            pltpu.SemaphoreType.DMA,
            pltpu.SemaphoreType.DMA,
        ],
    )
    def sc_embed(cat_h, pos_h, shp_h, wtab_h, ptab_h, stab_h, out_h,
                 cat0, cat1, pos0, pos1, shp0, shp1,
                 w0b, w1b, p0b, p1b, s0b, s1b,
                 sidx0, sidx1, srow0, srow1, sout0, sout1):
        wid = lax.axis_index("s") * NC + lax.axis_index("c")
        base = wid * tok_w
        slots = (
            (cat0, pos0, shp0, w0b, p0b, s0b, sidx0, srow0, sout0),
            (cat1, pos1, shp1, w1b, p1b, s1b, sidx1, srow1, sout1),
        )

        def issue_idx(g, sl):
            cat_v, pos_v, shp_v, _, _, _, sidx, _, _ = sl
            tok0 = base + g * CH
            pltpu.async_copy(cat_h.at[pl.ds(tok0, CH)], cat_v, sidx)
            pltpu.async_copy(pos_h.at[pl.ds(tok0, CH)], pos_v, sidx)
            pltpu.async_copy(shp_h.at[pl.ds(tok0, CH)], shp_v, sidx)

        def wait_idx(sl):
            cat_v, pos_v, shp_v, _, _, _, sidx, _, _ = sl
            pltpu.make_async_copy(cat_h.at[pl.ds(base, CH)], cat_v, sidx).wait()
            pltpu.make_async_copy(pos_h.at[pl.ds(base, CH)], pos_v, sidx).wait()
            pltpu.make_async_copy(shp_h.at[pl.ds(base, CH)], shp_v, sidx).wait()

        def issue_gathers(sl):
            cat_v, pos_v, shp_v, w_v, p_v, s_v, _, srow, _ = sl
            pltpu.async_copy(wtab_h.at[cat_v], w_v, srow)
            pltpu.async_copy(ptab_h.at[pos_v], p_v, srow)
            pltpu.async_copy(stab_h.at[shp_v], s_v, srow)

        def wait_gathers(sl):
            cat_v, pos_v, shp_v, w_v, p_v, s_v, _, srow, _ = sl
            pltpu.make_async_copy(wtab_h.at[cat_v], w_v, srow).wait()
            pltpu.make_async_copy(ptab_h.at[pos_v], p_v, srow).wait()
            pltpu.make_async_copy(stab_h.at[shp_v], s_v, srow).wait()

        def issue_out(g, sl):
            p_v, sout = sl[4], sl[8]
            tok0 = base + g * CH
            pltpu.async_copy(p_v, out_h.at[pl.ds(tok0, CH)], sout)

        def wait_out(sl):
            p_v, sout = sl[4], sl[8]
            pltpu.make_async_copy(p_v, out_h.at[pl.ds(base, CH)], sout).wait()

        def compute(sl):
            cat_v, _, _, w_v, p_v, s_v = sl[:6]

            def fix_grp(i, c):
                ids = cat_v[pl.ds(i * _L, _L)]
                msk = ids == 0
                nbad = jnp.max(msk.astype(jnp.int32))

                @pl.when(nbad > 0)
                def _():
                    rows = lax.iota(jnp.int32, _L) + i * _L
                    zeros = jnp.zeros((_L,), jnp.float32)
                    for col in range(D):
                        plsc.store_scatter(
                            w_v, [rows, jnp.full((_L,), col, jnp.int32)],
                            zeros, mask=msk)
                return c

            lax.fori_loop(0, CH // _L, fix_grp, 0)

            def tok_body(t, c):
                for j in range(D // _L):
                    sl2 = (t, pl.ds(j * _L, _L))
                    plsc.addupdate(p_v.at[sl2], w_v[sl2])
                    plsc.addupdate(p_v.at[sl2], s_v[sl2])
                return c

            lax.fori_loop(0, CH, tok_body, 0)

        issue_idx(0, slots[0])
        issue_idx(1, slots[1])
        wait_idx(slots[0])
        issue_gathers(slots[0])

        def outer(go, carry):
            for b in (0, 1):
                g = 2 * go + b
                cur, nxt = slots[b], slots[1 - b]

                @pl.when(g + 1 < n_chunks)
                def _():
                    wait_idx(nxt)

                    @pl.when(g >= 1)
                    def _():
                        wait_out(nxt)

                    issue_gathers(nxt)

                wait_gathers(cur)

                @pl.when(g + 2 < n_chunks)
                def _():
                    issue_idx(g + 2, cur)

                compute(cur)
                issue_out(g, cur)
            return carry

        lax.fori_loop(0, n_chunks // 2, outer, 0)
        wait_out(slots[0])
        wait_out(slots[1])

    return sc_embed


def kernel(cat_ids, position_ids, shape_ids, word_table, pos_table, shape_table):
    B, L = cat_ids.shape
    V, D = word_table.shape
    N = B * L
    sc_embed = _make_sc_embed(N, D)
    out = sc_embed(
        cat_ids.reshape(N),
        position_ids.reshape(N),
        shape_ids.reshape(N),
        word_table,
        pos_table,
        shape_table,
    )
    return out.reshape(B, L, D)
